# baseline (speedup 1.0000x reference)
import jax
import jax.numpy as jnp
from jax import lax
from jax.experimental import pallas as pl
from jax.experimental.pallas import tpu as pltpu
from jax.experimental.pallas import tpu_sc as plsc

N = 10000
NPAD = 10240
E = 320000
D_EMB = 128
D_HID = 128
D_OUT = 64
DH = 64

NC = 2
NS = 16
NW = NC * NS
ROWS_PER_TILE = NPAD // NS
SROWS_PER_TILE = N // NS

EB = 128
ENB = 80
ENBC = 40
EPAD = NW * ENB * EB

LB = 128
LNB = 50
LPAD = NW * LNB * LB
LPW = LNB * LB

_MESH = plsc.VectorSubcoreMesh(
    core_axis_name="c", subcore_axis_name="s", num_cores=NC, num_subcores=NS)
_SC_PARAMS = pltpu.CompilerParams(use_tc_tiling_on_sc=False)
_SC_PARAMS_NL = pltpu.CompilerParams(
    use_tc_tiling_on_sc=False, needs_layout_passes=False)


def _wid():
    return lax.axis_index("s") * NC + lax.axis_index("c")


def _deg_body(dst_hbm, ones_hbm, zeros_hbm, out_hbm, table, onesv, idxd2):
    cid = lax.axis_index("c")
    sid = lax.axis_index("s")
    wid = _wid()
    r0 = sid * ROWS_PER_TILE
    pltpu.sync_copy(zeros_hbm.at[pl.ds(r0, ROWS_PER_TILE)],
                    table.at[pl.ds(r0, ROWS_PER_TILE)])
    pltpu.sync_copy(ones_hbm, onesv)
    pltpu.sync_copy(dst_hbm.at[pl.ds(wid * ENB, ENB)], idxd2)
    plsc.subcore_barrier()

    def body(j, _):
        pltpu.sync_copy(onesv, table.at[idxd2.at[j]], add=True)
        return 0

    lax.fori_loop(0, ENB, body, 0)
    plsc.subcore_barrier()
    pltpu.sync_copy(table.at[pl.ds(r0, ROWS_PER_TILE)],
                    out_hbm.at[cid, pl.ds(r0, ROWS_PER_TILE)])


def _degrees(dst2):
    ones = jnp.ones((EB, 16), jnp.float32)
    zeros = jnp.zeros((NPAD, 16), jnp.float32)
    f = pl.kernel(
        _deg_body,
        out_type=jax.ShapeDtypeStruct((NC, NPAD, 16), jnp.float32),
        mesh=_MESH,
        compiler_params=_SC_PARAMS,
        scratch_types=[
            pltpu.VMEM_SHARED((NPAD, 16), jnp.float32),
            pltpu.VMEM((EB, 16), jnp.float32),
            pltpu.VMEM((ENB, EB), jnp.int32),
        ],
    )
    return f(dst2, ones, zeros)


def _agg_pass(hs_hbm, src_hbm, dst_hbm, zeros_hbm, write_out,
              table, hs_sp, idxs2, idxd2, rows, sems, scsems):
    sid = lax.axis_index("s")
    wid = _wid()
    r0 = sid * ROWS_PER_TILE
    h0 = sid * SROWS_PER_TILE
    pltpu.sync_copy(zeros_hbm.at[pl.ds(r0, ROWS_PER_TILE)],
                    table.at[pl.ds(r0, ROWS_PER_TILE)])
    pltpu.sync_copy(hs_hbm.at[pl.ds(h0, SROWS_PER_TILE)],
                    hs_sp.at[pl.ds(h0, SROWS_PER_TILE)])
    plsc.subcore_barrier()

    for c in range(ENB // ENBC):
        pltpu.sync_copy(src_hbm.at[pl.ds(wid * ENB + c * ENBC, ENBC)], idxs2)
        pltpu.sync_copy(dst_hbm.at[pl.ds(wid * ENB + c * ENBC, ENBC)], idxd2)
        for k in range(4):
            pltpu.async_copy(hs_sp.at[idxs2.at[k]], rows[k], sems[k])

        def body(t, _):
            j0 = 4 * t
            for k in range(4):
                pltpu.make_async_copy(
                    hs_sp.at[idxs2.at[j0 + k]], rows[k], sems[k]).wait()
                pltpu.async_copy(
                    rows[k], table.at[idxd2.at[j0 + k]], scsems[k], add=True)
            for k in range(4):
                @pl.when(j0 + k + 4 < ENBC)
                def _():
                    pltpu.make_async_copy(
                        rows[k], table.at[idxd2.at[j0 + k]], scsems[k]).wait()
                    pltpu.async_copy(
                        hs_sp.at[idxs2.at[j0 + k + 4]], rows[k], sems[k])

            return 0

        lax.fori_loop(0, ENBC // 4, body, 0)
        for k in range(4):
            pltpu.make_async_copy(
                rows[k], table.at[idxd2.at[ENBC - 4 + k]], scsems[k]).wait()
    plsc.subcore_barrier()
    write_out(table.at[pl.ds(r0, ROWS_PER_TILE)], r0)


def _agg2_body(hs_hbm, src_hbm, dst_hbm, zeros_hbm, out_hbm,
               table, hs_sp, idxs2, idxd2, r0b, r1b, r2b, r3b,
               s0, s1, s2, s3, c0, c1, c2, c3):
    cid = lax.axis_index("c")

    def write_out(tslice, r0):
        pltpu.sync_copy(tslice, out_hbm.at[cid, pl.ds(r0, ROWS_PER_TILE)])

    _agg_pass(hs_hbm, src_hbm, dst_hbm, zeros_hbm, write_out,
              table, hs_sp, idxs2, idxd2, [r0b, r1b, r2b, r3b],
              [s0, s1, s2, s3], [c0, c1, c2, c3])


def _aggab_body(hslo_hbm, hshi_hbm, src_hbm, dst_hbm, zeros_hbm, out_hbm,
                table, hs_sp, idxs2, idxd2, r0b, r1b, r2b, r3b,
                s0, s1, s2, s3, c0, c1, c2, c3):
    cid = lax.axis_index("c")
    for p, hs_hbm in enumerate((hslo_hbm, hshi_hbm)):
        def write_out(tslice, r0, p=p):
            pltpu.sync_copy(tslice, out_hbm.at[p, cid, pl.ds(r0, ROWS_PER_TILE)])

        _agg_pass(hs_hbm, src_hbm, dst_hbm, zeros_hbm, write_out,
                  table, hs_sp, idxs2, idxd2, [r0b, r1b, r2b, r3b],
                  [s0, s1, s2, s3], [c0, c1, c2, c3])
        if p == 0:
            plsc.subcore_barrier()


_AGG_SCRATCH = [
    pltpu.VMEM_SHARED((NPAD, DH), jnp.float32),
    pltpu.VMEM_SHARED((N, DH), jnp.float32),
    pltpu.VMEM((ENBC, EB), jnp.int32),
    pltpu.VMEM((ENBC, EB), jnp.int32),
    pltpu.VMEM((EB, DH), jnp.float32),
    pltpu.VMEM((EB, DH), jnp.float32),
    pltpu.VMEM((EB, DH), jnp.float32),
    pltpu.VMEM((EB, DH), jnp.float32),
    pltpu.SemaphoreType.DMA,
    pltpu.SemaphoreType.DMA,
    pltpu.SemaphoreType.DMA,
    pltpu.SemaphoreType.DMA,
    pltpu.SemaphoreType.DMA,
    pltpu.SemaphoreType.DMA,
    pltpu.SemaphoreType.DMA,
    pltpu.SemaphoreType.DMA,
]


def _aggregate(hs, src2, dst2):
    zeros = jnp.zeros((NPAD, DH), jnp.float32)
    f = pl.kernel(
        _agg2_body,
        out_type=jax.ShapeDtypeStruct((NC, NPAD, DH), jnp.float32),
        mesh=_MESH,
        compiler_params=_SC_PARAMS,
        scratch_types=list(_AGG_SCRATCH),
    )
    return f(hs, src2, dst2, zeros)


def _aggregate_ab(hs_lo, hs_hi, src2, dst2):
    zeros = jnp.zeros((NPAD, DH), jnp.float32)
    f = pl.kernel(
        _aggab_body,
        out_type=jax.ShapeDtypeStruct((2, NC, NPAD, DH), jnp.float32),
        mesh=_MESH,
        compiler_params=_SC_PARAMS,
        scratch_types=list(_AGG_SCRATCH),
    )
    return f(hs_lo, hs_hi, src2, dst2, zeros)


def _decode_body(z_hbm, sl_hbm, dl_hbm, out_hbm, z_sp, sidx2, didx2,
                 srows0, drows0, srows1, drows1, outv, vbuf,
                 sems0, semd0, sems1, semd1):
    sid = lax.axis_index("s")
    wid = _wid()
    h0 = sid * SROWS_PER_TILE
    pltpu.sync_copy(z_hbm.at[pl.ds(h0, SROWS_PER_TILE)],
                    z_sp.at[pl.ds(h0, SROWS_PER_TILE)])
    pltpu.sync_copy(sl_hbm.at[pl.ds(wid * LNB, LNB)], sidx2)
    pltpu.sync_copy(dl_hbm.at[pl.ds(wid * LNB, LNB)], didx2)
    plsc.subcore_barrier()

    pltpu.async_copy(z_sp.at[sidx2.at[0]], srows0, sems0)
    pltpu.async_copy(z_sp.at[didx2.at[0]], drows0, semd0)
    pltpu.async_copy(z_sp.at[sidx2.at[1]], srows1, sems1)
    pltpu.async_copy(z_sp.at[didx2.at[1]], drows1, semd1)

    iota = lax.iota(jnp.int32, 16)

    def compute(srows, drows):
        def grp(g, _):
            for u in range(16):
                lbl = g * 16 + u
                prods = [srows[lbl, pl.ds(16 * k, 16)] *
                         drows[lbl, pl.ds(16 * k, 16)] for k in range(4)]
                vbuf[u, pl.ds(0, 16)] = (prods[0] + prods[1]) + (prods[2] + prods[3])
            accs = [jnp.zeros((16,), jnp.float32) for _ in range(4)]
            for c in range(16):
                col = jnp.full((16,), c, jnp.int32)
                accs[c % 4] = accs[c % 4] + plsc.load_gather(vbuf, [iota, col])
            outv[pl.ds(g * 16, 16)] = (accs[0] + accs[1]) + (accs[2] + accs[3])
            return 0

        lax.fori_loop(0, LB // 16, grp, 0)

    def body(t, _):
        j0 = 2 * t
        pltpu.make_async_copy(z_sp.at[sidx2.at[j0]], srows0, sems0).wait()
        pltpu.make_async_copy(z_sp.at[didx2.at[j0]], drows0, semd0).wait()
        compute(srows0, drows0)
        pltpu.sync_copy(outv, out_hbm.at[pl.ds(wid * LPW + j0 * LB, LB)])

        @pl.when(j0 + 2 < LNB)
        def _():
            pltpu.async_copy(z_sp.at[sidx2.at[j0 + 2]], srows0, sems0)
            pltpu.async_copy(z_sp.at[didx2.at[j0 + 2]], drows0, semd0)

        pltpu.make_async_copy(z_sp.at[sidx2.at[j0 + 1]], srows1, sems1).wait()
        pltpu.make_async_copy(z_sp.at[didx2.at[j0 + 1]], drows1, semd1).wait()
        compute(srows1, drows1)
        pltpu.sync_copy(outv, out_hbm.at[pl.ds(wid * LPW + (j0 + 1) * LB, LB)])

        @pl.when(j0 + 3 < LNB)
        def _():
            pltpu.async_copy(z_sp.at[sidx2.at[j0 + 3]], srows1, sems1)
            pltpu.async_copy(z_sp.at[didx2.at[j0 + 3]], drows1, semd1)

        return 0

    lax.fori_loop(0, LNB // 2, body, 0)


def _decode(z, sl2, dl2):
    f = pl.kernel(
        _decode_body,
        out_type=jax.ShapeDtypeStruct((LPAD,), jnp.float32),
        mesh=_MESH,
        compiler_params=_SC_PARAMS_NL,
        scratch_types=[
            pltpu.VMEM_SHARED((N, D_OUT), jnp.float32),
            pltpu.VMEM((LNB, LB), jnp.int32),
            pltpu.VMEM((LNB, LB), jnp.int32),
            pltpu.VMEM((LB, D_OUT), jnp.float32),
            pltpu.VMEM((LB, D_OUT), jnp.float32),
            pltpu.VMEM((LB, D_OUT), jnp.float32),
            pltpu.VMEM((LB, D_OUT), jnp.float32),
            pltpu.VMEM((LB,), jnp.float32),
            pltpu.VMEM((16, 17), jnp.float32),
            pltpu.SemaphoreType.DMA,
            pltpu.SemaphoreType.DMA,
            pltpu.SemaphoreType.DMA,
            pltpu.SemaphoreType.DMA,
        ],
    )
    return f(z, sl2, dl2)


_BLK = 1000


def _t1_body(deg_ref, emb_ref, w1_ref, hlo_ref, hhi_ref, dinv_ref):
    deg = deg_ref[0, :, 0:1] + deg_ref[1, :, 0:1] + 1.0
    dinv = lax.rsqrt(deg)
    h = jnp.dot(emb_ref[...], w1_ref[...], preferred_element_type=jnp.float32)
    hs = h * dinv
    hlo_ref[...] = hs[:, :DH]
    hhi_ref[...] = hs[:, DH:]
    dinv_ref[...] = jnp.broadcast_to(dinv, (_BLK, 16))


def _t1(deg, emb, w1):
    return pl.pallas_call(
        _t1_body,
        grid=(N // _BLK,),
        in_specs=[
            pl.BlockSpec((2, _BLK, 16), lambda i: (0, i, 0)),
            pl.BlockSpec((_BLK, D_EMB), lambda i: (i, 0)),
            pl.BlockSpec((D_EMB, D_HID), lambda i: (0, 0)),
        ],
        out_specs=[
            pl.BlockSpec((_BLK, DH), lambda i: (i, 0)),
            pl.BlockSpec((_BLK, DH), lambda i: (i, 0)),
            pl.BlockSpec((_BLK, 16), lambda i: (i, 0)),
        ],
        out_shape=[
            jax.ShapeDtypeStruct((N, DH), jnp.float32),
            jax.ShapeDtypeStruct((N, DH), jnp.float32),
            jax.ShapeDtypeStruct((N, 16), jnp.float32),
        ],
    )(deg, emb, w1)


def _t2_body(p_ref, hlo_ref, hhi_ref, dinv_ref, b1_ref, w2_ref, hs2_ref):
    dinv = dinv_ref[:, 0:1]
    agg = jnp.concatenate(
        [p_ref[0, 0] + p_ref[0, 1] + hlo_ref[...],
         p_ref[1, 0] + p_ref[1, 1] + hhi_ref[...]], axis=1)
    x2 = jnp.maximum(agg * dinv + b1_ref[...], 0.0)
    h2 = jnp.dot(x2, w2_ref[...], preferred_element_type=jnp.float32)
    hs2_ref[...] = h2 * dinv


def _t2(p, hlo, hhi, dinv, b1, w2):
    return pl.pallas_call(
        _t2_body,
        grid=(N // _BLK,),
        in_specs=[
            pl.BlockSpec((2, 2, _BLK, DH), lambda i: (0, 0, i, 0)),
            pl.BlockSpec((_BLK, DH), lambda i: (i, 0)),
            pl.BlockSpec((_BLK, DH), lambda i: (i, 0)),
            pl.BlockSpec((_BLK, 16), lambda i: (i, 0)),
            pl.BlockSpec((1, D_HID), lambda i: (0, 0)),
            pl.BlockSpec((D_HID, D_OUT), lambda i: (0, 0)),
        ],
        out_specs=pl.BlockSpec((_BLK, D_OUT), lambda i: (i, 0)),
        out_shape=jax.ShapeDtypeStruct((N, D_OUT), jnp.float32),
    )(p, hlo, hhi, dinv, b1, w2)


def _t3_body(q_ref, hs2_ref, dinv_ref, b2_ref, z_ref):
    dinv = dinv_ref[:, 0:1]
    z_ref[...] = (q_ref[0] + q_ref[1] + hs2_ref[...]) * dinv + b2_ref[...]


def _t3(q, hs2, dinv, b2):
    return pl.pallas_call(
        _t3_body,
        grid=(N // _BLK,),
        in_specs=[
            pl.BlockSpec((2, _BLK, D_OUT), lambda i: (0, i, 0)),
            pl.BlockSpec((_BLK, D_OUT), lambda i: (i, 0)),
            pl.BlockSpec((_BLK, 16), lambda i: (i, 0)),
            pl.BlockSpec((1, D_OUT), lambda i: (0, 0)),
        ],
        out_specs=pl.BlockSpec((_BLK, D_OUT), lambda i: (i, 0)),
        out_shape=jax.ShapeDtypeStruct((N, D_OUT), jnp.float32),
    )(q, hs2, dinv, b2)


def kernel(edge_index, edge_label_index, embedding, W1, b1, W2, b2):
    epad = EPAD - E
    src2 = jnp.concatenate(
        [edge_index[0].astype(jnp.int32), jnp.zeros((epad,), jnp.int32)]
    ).reshape(NW * ENB, EB)
    dst2 = jnp.concatenate(
        [edge_index[1].astype(jnp.int32), jnp.full((epad,), N, jnp.int32)]
    ).reshape(NW * ENB, EB)

    nl = edge_label_index.shape[1]
    lpad = LPAD - nl
    sl2 = jnp.concatenate(
        [edge_label_index[0].astype(jnp.int32), jnp.zeros((lpad,), jnp.int32)]
    ).reshape(NW * LNB, LB)
    dl2 = jnp.concatenate(
        [edge_label_index[1].astype(jnp.int32), jnp.zeros((lpad,), jnp.int32)]
    ).reshape(NW * LNB, LB)

    deg = _degrees(dst2)
    hlo, hhi, dinv = _t1(deg, embedding, W1)
    p = _aggregate_ab(hlo, hhi, src2, dst2)
    hs2 = _t2(p, hlo, hhi, dinv, b1[None, :], W2)
    q = _aggregate(hs2, src2, dst2)
    z = _t3(q, hs2, dinv, b2[None, :])
    scores = _decode(z, sl2, dl2)
    return scores[:nl]


# speedup vs baseline: 1.0658x; 1.0658x over previous
import jax
import jax.numpy as jnp
from jax import lax
from jax.experimental import pallas as pl
from jax.experimental.pallas import tpu as pltpu
from jax.experimental.pallas import tpu_sc as plsc

N = 10000
NPAD = 10240
E = 320000
D_EMB = 128
D_HID = 128
D_OUT = 64
DH = 64

NC = 2
NS = 16
NW = NC * NS
ROWS_PER_TILE = NPAD // NS
SROWS_PER_TILE = N // NS

EB = 128
ENB = 80
ENBC = 40
EPAD = NW * ENB * EB

LB = 128
LNB = 50
LPAD = NW * LNB * LB
LPW = LNB * LB

_MESH = plsc.VectorSubcoreMesh(
    core_axis_name="c", subcore_axis_name="s", num_cores=NC, num_subcores=NS)
_SC_PARAMS = pltpu.CompilerParams(use_tc_tiling_on_sc=False)
_SC_PARAMS_NL = pltpu.CompilerParams(
    use_tc_tiling_on_sc=False, needs_layout_passes=False)


def _wid():
    return lax.axis_index("s") * NC + lax.axis_index("c")


def _deg_body(dst_hbm, ones_hbm, zeros_hbm, out_hbm, table, onesv, idxd2):
    cid = lax.axis_index("c")
    sid = lax.axis_index("s")
    wid = _wid()
    r0 = sid * ROWS_PER_TILE
    pltpu.sync_copy(zeros_hbm.at[pl.ds(r0, ROWS_PER_TILE)],
                    table.at[pl.ds(r0, ROWS_PER_TILE)])
    pltpu.sync_copy(ones_hbm, onesv)
    pltpu.sync_copy(dst_hbm.at[pl.ds(wid * ENB, ENB)], idxd2)
    plsc.subcore_barrier()

    def body(j, _):
        pltpu.sync_copy(onesv, table.at[idxd2.at[j]], add=True)
        return 0

    lax.fori_loop(0, ENB, body, 0)
    plsc.subcore_barrier()
    pltpu.sync_copy(table.at[pl.ds(r0, ROWS_PER_TILE)],
                    out_hbm.at[cid, pl.ds(r0, ROWS_PER_TILE)])


def _degrees(dst2):
    ones = jnp.ones((EB, 16), jnp.float32)
    zeros = jnp.zeros((NPAD, 16), jnp.float32)
    f = pl.kernel(
        _deg_body,
        out_type=jax.ShapeDtypeStruct((NC, NPAD, 16), jnp.float32),
        mesh=_MESH,
        compiler_params=_SC_PARAMS,
        scratch_types=[
            pltpu.VMEM_SHARED((NPAD, 16), jnp.float32),
            pltpu.VMEM((EB, 16), jnp.float32),
            pltpu.VMEM((ENB, EB), jnp.int32),
        ],
    )
    return f(dst2, ones, zeros)


def _agg_pass(hs_hbm, src_hbm, dst_hbm, zeros_hbm, write_out,
              table, hs_sp, idxs2, idxd2, rows, sems, scsems):
    sid = lax.axis_index("s")
    wid = _wid()
    r0 = sid * ROWS_PER_TILE
    h0 = sid * SROWS_PER_TILE
    pltpu.sync_copy(zeros_hbm.at[pl.ds(r0, ROWS_PER_TILE)],
                    table.at[pl.ds(r0, ROWS_PER_TILE)])
    pltpu.sync_copy(hs_hbm.at[pl.ds(h0, SROWS_PER_TILE)],
                    hs_sp.at[pl.ds(h0, SROWS_PER_TILE)])
    plsc.subcore_barrier()

    for c in range(ENB // ENBC):
        pltpu.sync_copy(src_hbm.at[pl.ds(wid * ENB + c * ENBC, ENBC)], idxs2)
        pltpu.sync_copy(dst_hbm.at[pl.ds(wid * ENB + c * ENBC, ENBC)], idxd2)
        for k in range(4):
            pltpu.async_copy(hs_sp.at[idxs2.at[k]], rows[k], sems[k])

        def body(t, _):
            j0 = 4 * t
            for k in range(4):
                pltpu.make_async_copy(
                    hs_sp.at[idxs2.at[j0 + k]], rows[k], sems[k]).wait()
                pltpu.sync_copy(rows[k], table.at[idxd2.at[j0 + k]], add=True)

                @pl.when(j0 + k + 4 < ENBC)
                def _():
                    pltpu.async_copy(
                        hs_sp.at[idxs2.at[j0 + k + 4]], rows[k], sems[k])

            return 0

        lax.fori_loop(0, ENBC // 4, body, 0)
    plsc.subcore_barrier()
    write_out(table.at[pl.ds(r0, ROWS_PER_TILE)], r0)


def _agg2_body(hs_hbm, src_hbm, dst_hbm, zeros_hbm, out_hbm,
               table, hs_sp, idxs2, idxd2, r0b, r1b, r2b, r3b,
               s0, s1, s2, s3, c0, c1, c2, c3):
    cid = lax.axis_index("c")

    def write_out(tslice, r0):
        pltpu.sync_copy(tslice, out_hbm.at[cid, pl.ds(r0, ROWS_PER_TILE)])

    _agg_pass(hs_hbm, src_hbm, dst_hbm, zeros_hbm, write_out,
              table, hs_sp, idxs2, idxd2, [r0b, r1b, r2b, r3b],
              [s0, s1, s2, s3], [c0, c1, c2, c3])


def _aggab_body(hslo_hbm, hshi_hbm, src_hbm, dst_hbm, zeros_hbm, out_hbm,
                table, hs_sp, idxs2, idxd2, r0b, r1b, r2b, r3b,
                s0, s1, s2, s3, c0, c1, c2, c3):
    cid = lax.axis_index("c")
    for p, hs_hbm in enumerate((hslo_hbm, hshi_hbm)):
        def write_out(tslice, r0, p=p):
            pltpu.sync_copy(tslice, out_hbm.at[p, cid, pl.ds(r0, ROWS_PER_TILE)])

        _agg_pass(hs_hbm, src_hbm, dst_hbm, zeros_hbm, write_out,
                  table, hs_sp, idxs2, idxd2, [r0b, r1b, r2b, r3b],
                  [s0, s1, s2, s3], [c0, c1, c2, c3])
        if p == 0:
            plsc.subcore_barrier()


_AGG_SCRATCH = [
    pltpu.VMEM_SHARED((NPAD, DH), jnp.float32),
    pltpu.VMEM_SHARED((N, DH), jnp.float32),
    pltpu.VMEM((ENBC, EB), jnp.int32),
    pltpu.VMEM((ENBC, EB), jnp.int32),
    pltpu.VMEM((EB, DH), jnp.float32),
    pltpu.VMEM((EB, DH), jnp.float32),
    pltpu.VMEM((EB, DH), jnp.float32),
    pltpu.VMEM((EB, DH), jnp.float32),
    pltpu.SemaphoreType.DMA,
    pltpu.SemaphoreType.DMA,
    pltpu.SemaphoreType.DMA,
    pltpu.SemaphoreType.DMA,
    pltpu.SemaphoreType.DMA,
    pltpu.SemaphoreType.DMA,
    pltpu.SemaphoreType.DMA,
    pltpu.SemaphoreType.DMA,
]


def _aggregate(hs, src2, dst2):
    zeros = jnp.zeros((NPAD, DH), jnp.float32)
    f = pl.kernel(
        _agg2_body,
        out_type=jax.ShapeDtypeStruct((NC, NPAD, DH), jnp.float32),
        mesh=_MESH,
        compiler_params=_SC_PARAMS,
        scratch_types=list(_AGG_SCRATCH),
    )
    return f(hs, src2, dst2, zeros)


def _aggregate_ab(hs_lo, hs_hi, src2, dst2):
    zeros = jnp.zeros((NPAD, DH), jnp.float32)
    f = pl.kernel(
        _aggab_body,
        out_type=jax.ShapeDtypeStruct((2, NC, NPAD, DH), jnp.float32),
        mesh=_MESH,
        compiler_params=_SC_PARAMS,
        scratch_types=list(_AGG_SCRATCH),
    )
    return f(hs_lo, hs_hi, src2, dst2, zeros)


def _decode_body(z_hbm, sl_hbm, dl_hbm, out_hbm, z_sp, sidx2, didx2,
                 srows0, drows0, srows1, drows1, outv, vbuf,
                 sems0, semd0, sems1, semd1):
    sid = lax.axis_index("s")
    wid = _wid()
    h0 = sid * SROWS_PER_TILE
    pltpu.sync_copy(z_hbm.at[pl.ds(h0, SROWS_PER_TILE)],
                    z_sp.at[pl.ds(h0, SROWS_PER_TILE)])
    pltpu.sync_copy(sl_hbm.at[pl.ds(wid * LNB, LNB)], sidx2)
    pltpu.sync_copy(dl_hbm.at[pl.ds(wid * LNB, LNB)], didx2)
    plsc.subcore_barrier()

    pltpu.async_copy(z_sp.at[sidx2.at[0]], srows0, sems0)
    pltpu.async_copy(z_sp.at[didx2.at[0]], drows0, semd0)
    pltpu.async_copy(z_sp.at[sidx2.at[1]], srows1, sems1)
    pltpu.async_copy(z_sp.at[didx2.at[1]], drows1, semd1)

    iota = lax.iota(jnp.int32, 16)

    def compute(srows, drows):
        def grp(g, _):
            for u in range(16):
                lbl = g * 16 + u
                prods = [srows[lbl, pl.ds(16 * k, 16)] *
                         drows[lbl, pl.ds(16 * k, 16)] for k in range(4)]
                vbuf[u, pl.ds(0, 16)] = (prods[0] + prods[1]) + (prods[2] + prods[3])
            accs = [jnp.zeros((16,), jnp.float32) for _ in range(4)]
            for c in range(16):
                col = jnp.full((16,), c, jnp.int32)
                accs[c % 4] = accs[c % 4] + plsc.load_gather(vbuf, [iota, col])
            outv[pl.ds(g * 16, 16)] = (accs[0] + accs[1]) + (accs[2] + accs[3])
            return 0

        lax.fori_loop(0, LB // 16, grp, 0)

    def body(t, _):
        j0 = 2 * t
        pltpu.make_async_copy(z_sp.at[sidx2.at[j0]], srows0, sems0).wait()
        pltpu.make_async_copy(z_sp.at[didx2.at[j0]], drows0, semd0).wait()
        compute(srows0, drows0)
        pltpu.sync_copy(outv, out_hbm.at[pl.ds(wid * LPW + j0 * LB, LB)])

        @pl.when(j0 + 2 < LNB)
        def _():
            pltpu.async_copy(z_sp.at[sidx2.at[j0 + 2]], srows0, sems0)
            pltpu.async_copy(z_sp.at[didx2.at[j0 + 2]], drows0, semd0)

        pltpu.make_async_copy(z_sp.at[sidx2.at[j0 + 1]], srows1, sems1).wait()
        pltpu.make_async_copy(z_sp.at[didx2.at[j0 + 1]], drows1, semd1).wait()
        compute(srows1, drows1)
        pltpu.sync_copy(outv, out_hbm.at[pl.ds(wid * LPW + (j0 + 1) * LB, LB)])

        @pl.when(j0 + 3 < LNB)
        def _():
            pltpu.async_copy(z_sp.at[sidx2.at[j0 + 3]], srows1, sems1)
            pltpu.async_copy(z_sp.at[didx2.at[j0 + 3]], drows1, semd1)

        return 0

    lax.fori_loop(0, LNB // 2, body, 0)


def _decode(z, sl2, dl2):
    f = pl.kernel(
        _decode_body,
        out_type=jax.ShapeDtypeStruct((LPAD,), jnp.float32),
        mesh=_MESH,
        compiler_params=_SC_PARAMS_NL,
        scratch_types=[
            pltpu.VMEM_SHARED((N, D_OUT), jnp.float32),
            pltpu.VMEM((LNB, LB), jnp.int32),
            pltpu.VMEM((LNB, LB), jnp.int32),
            pltpu.VMEM((LB, D_OUT), jnp.float32),
            pltpu.VMEM((LB, D_OUT), jnp.float32),
            pltpu.VMEM((LB, D_OUT), jnp.float32),
            pltpu.VMEM((LB, D_OUT), jnp.float32),
            pltpu.VMEM((LB,), jnp.float32),
            pltpu.VMEM((16, 17), jnp.float32),
            pltpu.SemaphoreType.DMA,
            pltpu.SemaphoreType.DMA,
            pltpu.SemaphoreType.DMA,
            pltpu.SemaphoreType.DMA,
        ],
    )
    return f(z, sl2, dl2)


_BLK = 1000


def _t1_body(deg_ref, emb_ref, w1_ref, hlo_ref, hhi_ref, dinv_ref):
    deg = deg_ref[0, :, 0:1] + deg_ref[1, :, 0:1] + 1.0
    dinv = lax.rsqrt(deg)
    h = jnp.dot(emb_ref[...], w1_ref[...], preferred_element_type=jnp.float32)
    hs = h * dinv
    hlo_ref[...] = hs[:, :DH]
    hhi_ref[...] = hs[:, DH:]
    dinv_ref[...] = jnp.broadcast_to(dinv, (_BLK, 16))


def _t1(deg, emb, w1):
    return pl.pallas_call(
        _t1_body,
        grid=(N // _BLK,),
        in_specs=[
            pl.BlockSpec((2, _BLK, 16), lambda i: (0, i, 0)),
            pl.BlockSpec((_BLK, D_EMB), lambda i: (i, 0)),
            pl.BlockSpec((D_EMB, D_HID), lambda i: (0, 0)),
        ],
        out_specs=[
            pl.BlockSpec((_BLK, DH), lambda i: (i, 0)),
            pl.BlockSpec((_BLK, DH), lambda i: (i, 0)),
            pl.BlockSpec((_BLK, 16), lambda i: (i, 0)),
        ],
        out_shape=[
            jax.ShapeDtypeStruct((N, DH), jnp.float32),
            jax.ShapeDtypeStruct((N, DH), jnp.float32),
            jax.ShapeDtypeStruct((N, 16), jnp.float32),
        ],
    )(deg, emb, w1)


def _t2_body(p_ref, hlo_ref, hhi_ref, dinv_ref, b1_ref, w2_ref, hs2_ref):
    dinv = dinv_ref[:, 0:1]
    agg = jnp.concatenate(
        [p_ref[0, 0] + p_ref[0, 1] + hlo_ref[...],
         p_ref[1, 0] + p_ref[1, 1] + hhi_ref[...]], axis=1)
    x2 = jnp.maximum(agg * dinv + b1_ref[...], 0.0)
    h2 = jnp.dot(x2, w2_ref[...], preferred_element_type=jnp.float32)
    hs2_ref[...] = h2 * dinv


def _t2(p, hlo, hhi, dinv, b1, w2):
    return pl.pallas_call(
        _t2_body,
        grid=(N // _BLK,),
        in_specs=[
            pl.BlockSpec((2, 2, _BLK, DH), lambda i: (0, 0, i, 0)),
            pl.BlockSpec((_BLK, DH), lambda i: (i, 0)),
            pl.BlockSpec((_BLK, DH), lambda i: (i, 0)),
            pl.BlockSpec((_BLK, 16), lambda i: (i, 0)),
            pl.BlockSpec((1, D_HID), lambda i: (0, 0)),
            pl.BlockSpec((D_HID, D_OUT), lambda i: (0, 0)),
        ],
        out_specs=pl.BlockSpec((_BLK, D_OUT), lambda i: (i, 0)),
        out_shape=jax.ShapeDtypeStruct((N, D_OUT), jnp.float32),
    )(p, hlo, hhi, dinv, b1, w2)


def _t3_body(q_ref, hs2_ref, dinv_ref, b2_ref, z_ref):
    dinv = dinv_ref[:, 0:1]
    z_ref[...] = (q_ref[0] + q_ref[1] + hs2_ref[...]) * dinv + b2_ref[...]


def _t3(q, hs2, dinv, b2):
    return pl.pallas_call(
        _t3_body,
        grid=(N // _BLK,),
        in_specs=[
            pl.BlockSpec((2, _BLK, D_OUT), lambda i: (0, i, 0)),
            pl.BlockSpec((_BLK, D_OUT), lambda i: (i, 0)),
            pl.BlockSpec((_BLK, 16), lambda i: (i, 0)),
            pl.BlockSpec((1, D_OUT), lambda i: (0, 0)),
        ],
        out_specs=pl.BlockSpec((_BLK, D_OUT), lambda i: (i, 0)),
        out_shape=jax.ShapeDtypeStruct((N, D_OUT), jnp.float32),
    )(q, hs2, dinv, b2)


def kernel(edge_index, edge_label_index, embedding, W1, b1, W2, b2):
    epad = EPAD - E
    src2 = jnp.concatenate(
        [edge_index[0].astype(jnp.int32), jnp.zeros((epad,), jnp.int32)]
    ).reshape(NW * ENB, EB)
    dst2 = jnp.concatenate(
        [edge_index[1].astype(jnp.int32), jnp.full((epad,), N, jnp.int32)]
    ).reshape(NW * ENB, EB)

    nl = edge_label_index.shape[1]
    lpad = LPAD - nl
    sl2 = jnp.concatenate(
        [edge_label_index[0].astype(jnp.int32), jnp.zeros((lpad,), jnp.int32)]
    ).reshape(NW * LNB, LB)
    dl2 = jnp.concatenate(
        [edge_label_index[1].astype(jnp.int32), jnp.zeros((lpad,), jnp.int32)]
    ).reshape(NW * LNB, LB)

    deg = _degrees(dst2)
    hlo, hhi, dinv = _t1(deg, embedding, W1)
    p = _aggregate_ab(hlo, hhi, src2, dst2)
    hs2 = _t2(p, hlo, hhi, dinv, b1[None, :], W2)
    q = _aggregate(hs2, src2, dst2)
    z = _t3(q, hs2, dinv, b2[None, :])
    scores = _decode(z, sl2, dl2)
    return scores[:nl]


# speedup vs baseline: 1.0887x; 1.0215x over previous
import jax
import jax.numpy as jnp
from jax import lax
from jax.experimental import pallas as pl
from jax.experimental.pallas import tpu as pltpu
from jax.experimental.pallas import tpu_sc as plsc

N = 10000
NPAD = 10240
E = 320000
D_EMB = 128
D_HID = 128
D_OUT = 64
DH = 64

NC = 2
NS = 16
NW = NC * NS
ROWS_PER_TILE = NPAD // NS
SROWS_PER_TILE = N // NS

EB = 128
ENB = 80
ENBC = 40
EPAD = NW * ENB * EB

LB = 128
LNB = 50
LPAD = NW * LNB * LB
LPW = LNB * LB

_MESH = plsc.VectorSubcoreMesh(
    core_axis_name="c", subcore_axis_name="s", num_cores=NC, num_subcores=NS)
_SC_PARAMS = pltpu.CompilerParams(use_tc_tiling_on_sc=False)
_SC_PARAMS_NL = pltpu.CompilerParams(
    use_tc_tiling_on_sc=False, needs_layout_passes=False)


def _wid():
    return lax.axis_index("s") * NC + lax.axis_index("c")


def _deg_body(dst_hbm, ones_hbm, zeros_hbm, out_hbm, table, onesv, idxd2):
    cid = lax.axis_index("c")
    sid = lax.axis_index("s")
    wid = _wid()
    r0 = sid * ROWS_PER_TILE
    pltpu.sync_copy(zeros_hbm.at[pl.ds(r0, ROWS_PER_TILE)],
                    table.at[pl.ds(r0, ROWS_PER_TILE)])
    pltpu.sync_copy(ones_hbm, onesv)
    pltpu.sync_copy(dst_hbm.at[pl.ds(wid * ENB, ENB)], idxd2)
    plsc.subcore_barrier()

    def body(j, _):
        pltpu.sync_copy(onesv, table.at[idxd2.at[j]], add=True)
        return 0

    lax.fori_loop(0, ENB, body, 0)
    plsc.subcore_barrier()
    pltpu.sync_copy(table.at[pl.ds(r0, ROWS_PER_TILE)],
                    out_hbm.at[cid, pl.ds(r0, ROWS_PER_TILE)])


def _degrees(dst2):
    ones = jnp.ones((EB, 16), jnp.float32)
    zeros = jnp.zeros((NPAD, 16), jnp.float32)
    f = pl.kernel(
        _deg_body,
        out_type=jax.ShapeDtypeStruct((NC, NPAD, 16), jnp.float32),
        mesh=_MESH,
        compiler_params=_SC_PARAMS,
        scratch_types=[
            pltpu.VMEM_SHARED((NPAD, 16), jnp.float32),
            pltpu.VMEM((EB, 16), jnp.float32),
            pltpu.VMEM((ENB, EB), jnp.int32),
        ],
    )
    return f(dst2, ones, zeros)


def _agg_pass(hs_hbm, src_hbm, dst_hbm, zeros_hbm, write_out,
              table, hs_sp, idxs2, idxd2, rows, sems, scsems):
    sid = lax.axis_index("s")
    wid = _wid()
    r0 = sid * ROWS_PER_TILE
    h0 = sid * SROWS_PER_TILE
    pltpu.sync_copy(zeros_hbm.at[pl.ds(r0, ROWS_PER_TILE)],
                    table.at[pl.ds(r0, ROWS_PER_TILE)])
    pltpu.sync_copy(hs_hbm.at[pl.ds(h0, SROWS_PER_TILE)],
                    hs_sp.at[pl.ds(h0, SROWS_PER_TILE)])
    plsc.subcore_barrier()

    for c in range(ENB // ENBC):
        pltpu.sync_copy(src_hbm.at[pl.ds(wid * ENB + c * ENBC, ENBC)], idxs2)
        pltpu.sync_copy(dst_hbm.at[pl.ds(wid * ENB + c * ENBC, ENBC)], idxd2)
        for k in range(4):
            pltpu.async_copy(hs_sp.at[idxs2.at[k]], rows[k], sems[k])

        def body(t, _):
            j0 = 4 * t
            for k in range(4):
                pltpu.make_async_copy(
                    hs_sp.at[idxs2.at[j0 + k]], rows[k], sems[k]).wait()
                pltpu.sync_copy(rows[k], table.at[idxd2.at[j0 + k]], add=True)

                @pl.when(j0 + k + 4 < ENBC)
                def _():
                    pltpu.async_copy(
                        hs_sp.at[idxs2.at[j0 + k + 4]], rows[k], sems[k])

            return 0

        lax.fori_loop(0, ENBC // 4, body, 0)
    plsc.subcore_barrier()
    write_out(table.at[pl.ds(r0, ROWS_PER_TILE)], r0)


def _agg2_body(hs_hbm, src_hbm, dst_hbm, zeros_hbm, out_hbm,
               table, hs_sp, idxs2, idxd2, r0b, r1b, r2b, r3b,
               s0, s1, s2, s3, c0, c1, c2, c3):
    cid = lax.axis_index("c")

    def write_out(tslice, r0):
        pltpu.sync_copy(tslice, out_hbm.at[cid, pl.ds(r0, ROWS_PER_TILE)])

    _agg_pass(hs_hbm, src_hbm, dst_hbm, zeros_hbm, write_out,
              table, hs_sp, idxs2, idxd2, [r0b, r1b, r2b, r3b],
              [s0, s1, s2, s3], [c0, c1, c2, c3])


def _aggab_body(hslo_hbm, hshi_hbm, src_hbm, dst_hbm, zeros_hbm, out_hbm,
                table, hs_sp, idxs2, idxd2, r0b, r1b, r2b, r3b,
                s0, s1, s2, s3, c0, c1, c2, c3):
    cid = lax.axis_index("c")
    for p, hs_hbm in enumerate((hslo_hbm, hshi_hbm)):
        def write_out(tslice, r0, p=p):
            pltpu.sync_copy(tslice, out_hbm.at[p, cid, pl.ds(r0, ROWS_PER_TILE)])

        _agg_pass(hs_hbm, src_hbm, dst_hbm, zeros_hbm, write_out,
                  table, hs_sp, idxs2, idxd2, [r0b, r1b, r2b, r3b],
                  [s0, s1, s2, s3], [c0, c1, c2, c3])
        if p == 0:
            plsc.subcore_barrier()


_AGG_SCRATCH = [
    pltpu.VMEM_SHARED((NPAD, DH), jnp.float32),
    pltpu.VMEM_SHARED((N, DH), jnp.float32),
    pltpu.VMEM((ENBC, EB), jnp.int32),
    pltpu.VMEM((ENBC, EB), jnp.int32),
    pltpu.VMEM((EB, DH), jnp.float32),
    pltpu.VMEM((EB, DH), jnp.float32),
    pltpu.VMEM((EB, DH), jnp.float32),
    pltpu.VMEM((EB, DH), jnp.float32),
    pltpu.SemaphoreType.DMA,
    pltpu.SemaphoreType.DMA,
    pltpu.SemaphoreType.DMA,
    pltpu.SemaphoreType.DMA,
    pltpu.SemaphoreType.DMA,
    pltpu.SemaphoreType.DMA,
    pltpu.SemaphoreType.DMA,
    pltpu.SemaphoreType.DMA,
]


def _aggregate(hs, src2, dst2):
    zeros = jnp.zeros((NPAD, DH), jnp.float32)
    f = pl.kernel(
        _agg2_body,
        out_type=jax.ShapeDtypeStruct((NC, NPAD, DH), jnp.float32),
        mesh=_MESH,
        compiler_params=_SC_PARAMS,
        scratch_types=list(_AGG_SCRATCH),
    )
    return f(hs, src2, dst2, zeros)


def _aggregate_ab(hs_lo, hs_hi, src2, dst2):
    zeros = jnp.zeros((NPAD, DH), jnp.float32)
    f = pl.kernel(
        _aggab_body,
        out_type=jax.ShapeDtypeStruct((2, NC, NPAD, DH), jnp.float32),
        mesh=_MESH,
        compiler_params=_SC_PARAMS,
        scratch_types=list(_AGG_SCRATCH),
    )
    return f(hs_lo, hs_hi, src2, dst2, zeros)


def _decode_body(z_hbm, sl_hbm, dl_hbm, out_hbm, z_sp, sidx2, didx2,
                 srows0, drows0, srows1, drows1, outv, vbuf,
                 sems0, semd0, sems1, semd1):
    sid = lax.axis_index("s")
    wid = _wid()
    h0 = sid * SROWS_PER_TILE
    pltpu.sync_copy(z_hbm.at[pl.ds(h0, SROWS_PER_TILE)],
                    z_sp.at[pl.ds(h0, SROWS_PER_TILE)])
    pltpu.sync_copy(sl_hbm.at[pl.ds(wid * LNB, LNB)], sidx2)
    pltpu.sync_copy(dl_hbm.at[pl.ds(wid * LNB, LNB)], didx2)
    plsc.subcore_barrier()

    pltpu.async_copy(z_sp.at[sidx2.at[0]], srows0, sems0)
    pltpu.async_copy(z_sp.at[didx2.at[0]], drows0, semd0)
    pltpu.async_copy(z_sp.at[sidx2.at[1]], srows1, sems1)
    pltpu.async_copy(z_sp.at[didx2.at[1]], drows1, semd1)

    iota = lax.iota(jnp.int32, 16)

    def compute(srows, drows):
        def grp(g, _):
            for u in range(16):
                lbl = g * 16 + u
                halves = []
                for k in range(2):
                    p = (srows[lbl, pl.ds(32 * k, 32)] *
                         drows[lbl, pl.ds(32 * k, 32)])
                    a, b = plsc.unpack(p, format=plsc.PackFormat.INTERLEAVED)
                    halves.append(a + b)
                vbuf[u, pl.ds(0, 16)] = halves[0] + halves[1]
            accs = [jnp.zeros((16,), jnp.float32) for _ in range(4)]
            for c in range(16):
                col = jnp.full((16,), c, jnp.int32)
                accs[c % 4] = accs[c % 4] + plsc.load_gather(vbuf, [iota, col])
            outv[pl.ds(g * 16, 16)] = (accs[0] + accs[1]) + (accs[2] + accs[3])
            return 0

        lax.fori_loop(0, LB // 16, grp, 0)

    def body(t, _):
        j0 = 2 * t
        pltpu.make_async_copy(z_sp.at[sidx2.at[j0]], srows0, sems0).wait()
        pltpu.make_async_copy(z_sp.at[didx2.at[j0]], drows0, semd0).wait()
        compute(srows0, drows0)
        pltpu.sync_copy(outv, out_hbm.at[pl.ds(wid * LPW + j0 * LB, LB)])

        @pl.when(j0 + 2 < LNB)
        def _():
            pltpu.async_copy(z_sp.at[sidx2.at[j0 + 2]], srows0, sems0)
            pltpu.async_copy(z_sp.at[didx2.at[j0 + 2]], drows0, semd0)

        pltpu.make_async_copy(z_sp.at[sidx2.at[j0 + 1]], srows1, sems1).wait()
        pltpu.make_async_copy(z_sp.at[didx2.at[j0 + 1]], drows1, semd1).wait()
        compute(srows1, drows1)
        pltpu.sync_copy(outv, out_hbm.at[pl.ds(wid * LPW + (j0 + 1) * LB, LB)])

        @pl.when(j0 + 3 < LNB)
        def _():
            pltpu.async_copy(z_sp.at[sidx2.at[j0 + 3]], srows1, sems1)
            pltpu.async_copy(z_sp.at[didx2.at[j0 + 3]], drows1, semd1)

        return 0

    lax.fori_loop(0, LNB // 2, body, 0)


def _decode(z, sl2, dl2):
    f = pl.kernel(
        _decode_body,
        out_type=jax.ShapeDtypeStruct((LPAD,), jnp.float32),
        mesh=_MESH,
        compiler_params=_SC_PARAMS_NL,
        scratch_types=[
            pltpu.VMEM_SHARED((N, D_OUT), jnp.bfloat16),
            pltpu.VMEM((LNB, LB), jnp.int32),
            pltpu.VMEM((LNB, LB), jnp.int32),
            pltpu.VMEM((LB, D_OUT), jnp.bfloat16),
            pltpu.VMEM((LB, D_OUT), jnp.bfloat16),
            pltpu.VMEM((LB, D_OUT), jnp.bfloat16),
            pltpu.VMEM((LB, D_OUT), jnp.bfloat16),
            pltpu.VMEM((LB,), jnp.float32),
            pltpu.VMEM((16, 17), jnp.float32),
            pltpu.SemaphoreType.DMA,
            pltpu.SemaphoreType.DMA,
            pltpu.SemaphoreType.DMA,
            pltpu.SemaphoreType.DMA,
        ],
    )
    return f(z, sl2, dl2)


_BLK = 1000


def _t1_body(deg_ref, emb_ref, w1_ref, hlo_ref, hhi_ref, dinv_ref):
    deg = deg_ref[0, :, 0:1] + deg_ref[1, :, 0:1] + 1.0
    dinv = lax.rsqrt(deg)
    h = jnp.dot(emb_ref[...], w1_ref[...], preferred_element_type=jnp.float32)
    hs = h * dinv
    hlo_ref[...] = hs[:, :DH]
    hhi_ref[...] = hs[:, DH:]
    dinv_ref[...] = jnp.broadcast_to(dinv, (_BLK, 16))


def _t1(deg, emb, w1):
    return pl.pallas_call(
        _t1_body,
        grid=(N // _BLK,),
        in_specs=[
            pl.BlockSpec((2, _BLK, 16), lambda i: (0, i, 0)),
            pl.BlockSpec((_BLK, D_EMB), lambda i: (i, 0)),
            pl.BlockSpec((D_EMB, D_HID), lambda i: (0, 0)),
        ],
        out_specs=[
            pl.BlockSpec((_BLK, DH), lambda i: (i, 0)),
            pl.BlockSpec((_BLK, DH), lambda i: (i, 0)),
            pl.BlockSpec((_BLK, 16), lambda i: (i, 0)),
        ],
        out_shape=[
            jax.ShapeDtypeStruct((N, DH), jnp.float32),
            jax.ShapeDtypeStruct((N, DH), jnp.float32),
            jax.ShapeDtypeStruct((N, 16), jnp.float32),
        ],
    )(deg, emb, w1)


def _t2_body(p_ref, hlo_ref, hhi_ref, dinv_ref, b1_ref, w2_ref, hs2_ref):
    dinv = dinv_ref[:, 0:1]
    agg = jnp.concatenate(
        [p_ref[0, 0] + p_ref[0, 1] + hlo_ref[...],
         p_ref[1, 0] + p_ref[1, 1] + hhi_ref[...]], axis=1)
    x2 = jnp.maximum(agg * dinv + b1_ref[...], 0.0)
    h2 = jnp.dot(x2, w2_ref[...], preferred_element_type=jnp.float32)
    hs2_ref[...] = h2 * dinv


def _t2(p, hlo, hhi, dinv, b1, w2):
    return pl.pallas_call(
        _t2_body,
        grid=(N // _BLK,),
        in_specs=[
            pl.BlockSpec((2, 2, _BLK, DH), lambda i: (0, 0, i, 0)),
            pl.BlockSpec((_BLK, DH), lambda i: (i, 0)),
            pl.BlockSpec((_BLK, DH), lambda i: (i, 0)),
            pl.BlockSpec((_BLK, 16), lambda i: (i, 0)),
            pl.BlockSpec((1, D_HID), lambda i: (0, 0)),
            pl.BlockSpec((D_HID, D_OUT), lambda i: (0, 0)),
        ],
        out_specs=pl.BlockSpec((_BLK, D_OUT), lambda i: (i, 0)),
        out_shape=jax.ShapeDtypeStruct((N, D_OUT), jnp.float32),
    )(p, hlo, hhi, dinv, b1, w2)


def _t3_body(q_ref, hs2_ref, dinv_ref, b2_ref, z_ref):
    dinv = dinv_ref[:, 0:1]
    z = (q_ref[0] + q_ref[1] + hs2_ref[...]) * dinv + b2_ref[...]
    z_ref[...] = z.astype(jnp.bfloat16)


def _t3(q, hs2, dinv, b2):
    return pl.pallas_call(
        _t3_body,
        grid=(N // _BLK,),
        in_specs=[
            pl.BlockSpec((2, _BLK, D_OUT), lambda i: (0, i, 0)),
            pl.BlockSpec((_BLK, D_OUT), lambda i: (i, 0)),
            pl.BlockSpec((_BLK, 16), lambda i: (i, 0)),
            pl.BlockSpec((1, D_OUT), lambda i: (0, 0)),
        ],
        out_specs=pl.BlockSpec((_BLK, D_OUT), lambda i: (i, 0)),
        out_shape=jax.ShapeDtypeStruct((N, D_OUT), jnp.bfloat16),
    )(q, hs2, dinv, b2)


def kernel(edge_index, edge_label_index, embedding, W1, b1, W2, b2):
    epad = EPAD - E
    src2 = jnp.concatenate(
        [edge_index[0].astype(jnp.int32), jnp.zeros((epad,), jnp.int32)]
    ).reshape(NW * ENB, EB)
    dst2 = jnp.concatenate(
        [edge_index[1].astype(jnp.int32), jnp.full((epad,), N, jnp.int32)]
    ).reshape(NW * ENB, EB)

    nl = edge_label_index.shape[1]
    lpad = LPAD - nl
    sl2 = jnp.concatenate(
        [edge_label_index[0].astype(jnp.int32), jnp.zeros((lpad,), jnp.int32)]
    ).reshape(NW * LNB, LB)
    dl2 = jnp.concatenate(
        [edge_label_index[1].astype(jnp.int32), jnp.zeros((lpad,), jnp.int32)]
    ).reshape(NW * LNB, LB)

    deg = _degrees(dst2)
    hlo, hhi, dinv = _t1(deg, embedding, W1)
    p = _aggregate_ab(hlo, hhi, src2, dst2)
    hs2 = _t2(p, hlo, hhi, dinv, b1[None, :], W2)
    q = _aggregate(hs2, src2, dst2)
    z = _t3(q, hs2, dinv, b2[None, :])
    scores = _decode(z, sl2, dl2)
    return scores[:nl]


# speedup vs baseline: 1.0947x; 1.0055x over previous
import jax
import jax.numpy as jnp
from jax import lax
from jax.experimental import pallas as pl
from jax.experimental.pallas import tpu as pltpu
from jax.experimental.pallas import tpu_sc as plsc

N = 10000
NPAD = 10240
E = 320000
D_EMB = 128
D_HID = 128
D_OUT = 64
DH = 64

NC = 2
NS = 16
NW = NC * NS
ROWS_PER_TILE = NPAD // NS
SROWS_PER_TILE = N // NS

EB = 128
ENB = 80
ENBC = 40
EPAD = NW * ENB * EB

LB = 128
LNB = 50
LPAD = NW * LNB * LB
LPW = LNB * LB

_MESH = plsc.VectorSubcoreMesh(
    core_axis_name="c", subcore_axis_name="s", num_cores=NC, num_subcores=NS)
_SC_PARAMS = pltpu.CompilerParams(use_tc_tiling_on_sc=False)
_SC_PARAMS_NL = pltpu.CompilerParams(
    use_tc_tiling_on_sc=False, needs_layout_passes=False)


def _wid():
    return lax.axis_index("s") * NC + lax.axis_index("c")


def _deg_body(dst_hbm, ones_hbm, zeros_hbm, out_hbm, table, onesv, idxd2):
    cid = lax.axis_index("c")
    sid = lax.axis_index("s")
    wid = _wid()
    r0 = sid * ROWS_PER_TILE
    pltpu.sync_copy(zeros_hbm.at[pl.ds(r0, ROWS_PER_TILE)],
                    table.at[pl.ds(r0, ROWS_PER_TILE)])
    pltpu.sync_copy(ones_hbm, onesv)
    pltpu.sync_copy(dst_hbm.at[pl.ds(wid * ENB, ENB)], idxd2)
    plsc.subcore_barrier()

    def body(j, _):
        pltpu.sync_copy(onesv, table.at[idxd2.at[j]], add=True)
        return 0

    lax.fori_loop(0, ENB, body, 0)
    plsc.subcore_barrier()
    pltpu.sync_copy(table.at[pl.ds(r0, ROWS_PER_TILE)],
                    out_hbm.at[cid, pl.ds(r0, ROWS_PER_TILE)])


def _degrees(dst2):
    ones = jnp.ones((EB, 16), jnp.float32)
    zeros = jnp.zeros((NPAD, 16), jnp.float32)
    f = pl.kernel(
        _deg_body,
        out_type=jax.ShapeDtypeStruct((NC, NPAD, 16), jnp.float32),
        mesh=_MESH,
        compiler_params=_SC_PARAMS,
        scratch_types=[
            pltpu.VMEM_SHARED((NPAD, 16), jnp.float32),
            pltpu.VMEM((EB, 16), jnp.float32),
            pltpu.VMEM((ENB, EB), jnp.int32),
        ],
    )
    return f(dst2, ones, zeros)


def _agg_pass(hs_hbm, src_hbm, dst_hbm, zeros_hbm, write_out,
              table, hs_sp, idxs2, idxd2, rows, sems):
    sid = lax.axis_index("s")
    wid = _wid()
    r0 = sid * ROWS_PER_TILE
    h0 = sid * SROWS_PER_TILE
    pltpu.sync_copy(zeros_hbm.at[pl.ds(r0, ROWS_PER_TILE)],
                    table.at[pl.ds(r0, ROWS_PER_TILE)])
    pltpu.sync_copy(hs_hbm.at[pl.ds(h0, SROWS_PER_TILE)],
                    hs_sp.at[pl.ds(h0, SROWS_PER_TILE)])
    plsc.subcore_barrier()

    for c in range(ENB // ENBC):
        pltpu.sync_copy(src_hbm.at[pl.ds(wid * ENB + c * ENBC, ENBC)], idxs2)
        pltpu.sync_copy(dst_hbm.at[pl.ds(wid * ENB + c * ENBC, ENBC)], idxd2)
        for k in range(4):
            pltpu.async_copy(hs_sp.at[idxs2.at[k]], rows[k], sems[k])

        def body(t, _):
            j0 = 4 * t
            for k in range(4):
                pltpu.make_async_copy(
                    hs_sp.at[idxs2.at[j0 + k]], rows[k], sems[k]).wait()
                pltpu.sync_copy(rows[k], table.at[idxd2.at[j0 + k]], add=True)

                @pl.when(j0 + k + 4 < ENBC)
                def _():
                    pltpu.async_copy(
                        hs_sp.at[idxs2.at[j0 + k + 4]], rows[k], sems[k])

            return 0

        lax.fori_loop(0, ENBC // 4, body, 0)
    plsc.subcore_barrier()
    write_out(table.at[pl.ds(r0, ROWS_PER_TILE)], r0)


def _agg2_body(hs_hbm, src_hbm, dst_hbm, zeros_hbm, out_hbm,
               table, hs_sp, idxs2, idxd2, r0b, r1b, r2b, r3b,
               s0, s1, s2, s3):
    cid = lax.axis_index("c")

    def write_out(tslice, r0):
        pltpu.sync_copy(tslice, out_hbm.at[cid, pl.ds(r0, ROWS_PER_TILE)])

    _agg_pass(hs_hbm, src_hbm, dst_hbm, zeros_hbm, write_out,
              table, hs_sp, idxs2, idxd2, [r0b, r1b, r2b, r3b],
              [s0, s1, s2, s3])


def _aggab_body(hslo_hbm, hshi_hbm, src_hbm, dst_hbm, zeros_hbm, out_hbm,
                table, hs_sp, idxs2, idxd2, r0b, r1b, r2b, r3b,
                s0, s1, s2, s3):
    cid = lax.axis_index("c")
    for p, hs_hbm in enumerate((hslo_hbm, hshi_hbm)):
        def write_out(tslice, r0, p=p):
            pltpu.sync_copy(tslice, out_hbm.at[p, cid, pl.ds(r0, ROWS_PER_TILE)])

        _agg_pass(hs_hbm, src_hbm, dst_hbm, zeros_hbm, write_out,
                  table, hs_sp, idxs2, idxd2, [r0b, r1b, r2b, r3b],
                  [s0, s1, s2, s3])
        if p == 0:
            plsc.subcore_barrier()


_AGG_SCRATCH = [
    pltpu.VMEM_SHARED((NPAD, DH), jnp.float32),
    pltpu.VMEM_SHARED((N, DH), jnp.float32),
    pltpu.VMEM((ENBC, EB), jnp.int32),
    pltpu.VMEM((ENBC, EB), jnp.int32),
    pltpu.VMEM((EB, DH), jnp.float32),
    pltpu.VMEM((EB, DH), jnp.float32),
    pltpu.VMEM((EB, DH), jnp.float32),
    pltpu.VMEM((EB, DH), jnp.float32),
    pltpu.SemaphoreType.DMA,
    pltpu.SemaphoreType.DMA,
    pltpu.SemaphoreType.DMA,
    pltpu.SemaphoreType.DMA,
]


def _aggregate(hs, src2, dst2):
    zeros = jnp.zeros((NPAD, DH), jnp.float32)
    f = pl.kernel(
        _agg2_body,
        out_type=jax.ShapeDtypeStruct((NC, NPAD, DH), jnp.float32),
        mesh=_MESH,
        compiler_params=_SC_PARAMS,
        scratch_types=list(_AGG_SCRATCH),
    )
    return f(hs, src2, dst2, zeros)


def _aggregate_ab(hs_lo, hs_hi, src2, dst2):
    zeros = jnp.zeros((NPAD, DH), jnp.float32)
    f = pl.kernel(
        _aggab_body,
        out_type=jax.ShapeDtypeStruct((2, NC, NPAD, DH), jnp.float32),
        mesh=_MESH,
        compiler_params=_SC_PARAMS,
        scratch_types=list(_AGG_SCRATCH),
    )
    return f(hs_lo, hs_hi, src2, dst2, zeros)


def _decode_body(z_hbm, sl_hbm, dl_hbm, out_hbm, z_sp, sidx2, didx2,
                 srows0, drows0, srows1, drows1, outv0, outv1, vbuf,
                 sems0, semd0, sems1, semd1, semo0, semo1):
    sid = lax.axis_index("s")
    wid = _wid()
    h0 = sid * SROWS_PER_TILE
    pltpu.sync_copy(z_hbm.at[pl.ds(h0, SROWS_PER_TILE)],
                    z_sp.at[pl.ds(h0, SROWS_PER_TILE)])
    pltpu.sync_copy(sl_hbm.at[pl.ds(wid * LNB, LNB)], sidx2)
    pltpu.sync_copy(dl_hbm.at[pl.ds(wid * LNB, LNB)], didx2)
    plsc.subcore_barrier()

    pltpu.async_copy(z_sp.at[sidx2.at[0]], srows0, sems0)
    pltpu.async_copy(z_sp.at[didx2.at[0]], drows0, semd0)
    pltpu.async_copy(z_sp.at[sidx2.at[1]], srows1, sems1)
    pltpu.async_copy(z_sp.at[didx2.at[1]], drows1, semd1)

    iota = lax.iota(jnp.int32, 16)

    def compute(srows, drows, outv):
        def grp(g, _):
            for u in range(16):
                lbl = g * 16 + u
                halves = []
                for k in range(2):
                    p = (srows[lbl, pl.ds(32 * k, 32)] *
                         drows[lbl, pl.ds(32 * k, 32)])
                    a, b = plsc.unpack(p, format=plsc.PackFormat.INTERLEAVED)
                    halves.append(a + b)
                vbuf[u, pl.ds(0, 16)] = halves[0] + halves[1]
            accs = [jnp.zeros((16,), jnp.float32) for _ in range(4)]
            for c in range(16):
                col = jnp.full((16,), c, jnp.int32)
                accs[c % 4] = accs[c % 4] + plsc.load_gather(vbuf, [iota, col])
            outv[pl.ds(g * 16, 16)] = (accs[0] + accs[1]) + (accs[2] + accs[3])
            return 0

        lax.fori_loop(0, LB // 16, grp, 0)

    def body(t, _):
        j0 = 2 * t
        pltpu.make_async_copy(z_sp.at[sidx2.at[j0]], srows0, sems0).wait()
        pltpu.make_async_copy(z_sp.at[didx2.at[j0]], drows0, semd0).wait()

        @pl.when(j0 >= 2)
        def _():
            pltpu.make_async_copy(
                outv0, out_hbm.at[pl.ds(wid * LPW, LB)], semo0).wait()

        compute(srows0, drows0, outv0)
        pltpu.async_copy(outv0, out_hbm.at[pl.ds(wid * LPW + j0 * LB, LB)],
                         semo0)

        @pl.when(j0 + 2 < LNB)
        def _():
            pltpu.async_copy(z_sp.at[sidx2.at[j0 + 2]], srows0, sems0)
            pltpu.async_copy(z_sp.at[didx2.at[j0 + 2]], drows0, semd0)

        pltpu.make_async_copy(z_sp.at[sidx2.at[j0 + 1]], srows1, sems1).wait()
        pltpu.make_async_copy(z_sp.at[didx2.at[j0 + 1]], drows1, semd1).wait()

        @pl.when(j0 >= 2)
        def _():
            pltpu.make_async_copy(
                outv1, out_hbm.at[pl.ds(wid * LPW, LB)], semo1).wait()

        compute(srows1, drows1, outv1)
        pltpu.async_copy(outv1, out_hbm.at[pl.ds(wid * LPW + (j0 + 1) * LB, LB)],
                         semo1)

        @pl.when(j0 + 3 < LNB)
        def _():
            pltpu.async_copy(z_sp.at[sidx2.at[j0 + 3]], srows1, sems1)
            pltpu.async_copy(z_sp.at[didx2.at[j0 + 3]], drows1, semd1)

        return 0

    lax.fori_loop(0, LNB // 2, body, 0)
    pltpu.make_async_copy(outv0, out_hbm.at[pl.ds(wid * LPW, LB)], semo0).wait()
    pltpu.make_async_copy(outv1, out_hbm.at[pl.ds(wid * LPW, LB)], semo1).wait()


def _decode(z, sl2, dl2):
    f = pl.kernel(
        _decode_body,
        out_type=jax.ShapeDtypeStruct((LPAD,), jnp.float32),
        mesh=_MESH,
        compiler_params=_SC_PARAMS_NL,
        scratch_types=[
            pltpu.VMEM_SHARED((N, D_OUT), jnp.bfloat16),
            pltpu.VMEM((LNB, LB), jnp.int32),
            pltpu.VMEM((LNB, LB), jnp.int32),
            pltpu.VMEM((LB, D_OUT), jnp.bfloat16),
            pltpu.VMEM((LB, D_OUT), jnp.bfloat16),
            pltpu.VMEM((LB, D_OUT), jnp.bfloat16),
            pltpu.VMEM((LB, D_OUT), jnp.bfloat16),
            pltpu.VMEM((LB,), jnp.float32),
            pltpu.VMEM((LB,), jnp.float32),
            pltpu.VMEM((16, 17), jnp.float32),
            pltpu.SemaphoreType.DMA,
            pltpu.SemaphoreType.DMA,
            pltpu.SemaphoreType.DMA,
            pltpu.SemaphoreType.DMA,
            pltpu.SemaphoreType.DMA,
            pltpu.SemaphoreType.DMA,
        ],
    )
    return f(z, sl2, dl2)


_BLK = 1000


def _t1_body(deg_ref, emb_ref, w1_ref, hlo_ref, hhi_ref, dinv_ref):
    deg = deg_ref[0, :, 0:1] + deg_ref[1, :, 0:1] + 1.0
    dinv = lax.rsqrt(deg)
    h = jnp.dot(emb_ref[...], w1_ref[...], preferred_element_type=jnp.float32)
    hs = h * dinv
    hlo_ref[...] = hs[:, :DH]
    hhi_ref[...] = hs[:, DH:]
    dinv_ref[...] = jnp.broadcast_to(dinv, (_BLK, 16))


def _t1(deg, emb, w1):
    return pl.pallas_call(
        _t1_body,
        grid=(N // _BLK,),
        in_specs=[
            pl.BlockSpec((2, _BLK, 16), lambda i: (0, i, 0)),
            pl.BlockSpec((_BLK, D_EMB), lambda i: (i, 0)),
            pl.BlockSpec((D_EMB, D_HID), lambda i: (0, 0)),
        ],
        out_specs=[
            pl.BlockSpec((_BLK, DH), lambda i: (i, 0)),
            pl.BlockSpec((_BLK, DH), lambda i: (i, 0)),
            pl.BlockSpec((_BLK, 16), lambda i: (i, 0)),
        ],
        out_shape=[
            jax.ShapeDtypeStruct((N, DH), jnp.float32),
            jax.ShapeDtypeStruct((N, DH), jnp.float32),
            jax.ShapeDtypeStruct((N, 16), jnp.float32),
        ],
    )(deg, emb, w1)


def _t2_body(p_ref, hlo_ref, hhi_ref, dinv_ref, b1_ref, w2_ref, hs2_ref):
    dinv = dinv_ref[:, 0:1]
    agg = jnp.concatenate(
        [p_ref[0, 0] + p_ref[0, 1] + hlo_ref[...],
         p_ref[1, 0] + p_ref[1, 1] + hhi_ref[...]], axis=1)
    x2 = jnp.maximum(agg * dinv + b1_ref[...], 0.0)
    h2 = jnp.dot(x2, w2_ref[...], preferred_element_type=jnp.float32)
    hs2_ref[...] = h2 * dinv


def _t2(p, hlo, hhi, dinv, b1, w2):
    return pl.pallas_call(
        _t2_body,
        grid=(N // _BLK,),
        in_specs=[
            pl.BlockSpec((2, 2, _BLK, DH), lambda i: (0, 0, i, 0)),
            pl.BlockSpec((_BLK, DH), lambda i: (i, 0)),
            pl.BlockSpec((_BLK, DH), lambda i: (i, 0)),
            pl.BlockSpec((_BLK, 16), lambda i: (i, 0)),
            pl.BlockSpec((1, D_HID), lambda i: (0, 0)),
            pl.BlockSpec((D_HID, D_OUT), lambda i: (0, 0)),
        ],
        out_specs=pl.BlockSpec((_BLK, D_OUT), lambda i: (i, 0)),
        out_shape=jax.ShapeDtypeStruct((N, D_OUT), jnp.float32),
    )(p, hlo, hhi, dinv, b1, w2)


def _t3_body(q_ref, hs2_ref, dinv_ref, b2_ref, z_ref):
    dinv = dinv_ref[:, 0:1]
    z = (q_ref[0] + q_ref[1] + hs2_ref[...]) * dinv + b2_ref[...]
    z_ref[...] = z.astype(jnp.bfloat16)


def _t3(q, hs2, dinv, b2):
    return pl.pallas_call(
        _t3_body,
        grid=(N // _BLK,),
        in_specs=[
            pl.BlockSpec((2, _BLK, D_OUT), lambda i: (0, i, 0)),
            pl.BlockSpec((_BLK, D_OUT), lambda i: (i, 0)),
            pl.BlockSpec((_BLK, 16), lambda i: (i, 0)),
            pl.BlockSpec((1, D_OUT), lambda i: (0, 0)),
        ],
        out_specs=pl.BlockSpec((_BLK, D_OUT), lambda i: (i, 0)),
        out_shape=jax.ShapeDtypeStruct((N, D_OUT), jnp.bfloat16),
    )(q, hs2, dinv, b2)


def kernel(edge_index, edge_label_index, embedding, W1, b1, W2, b2):
    epad = EPAD - E
    src2 = jnp.concatenate(
        [edge_index[0].astype(jnp.int32), jnp.zeros((epad,), jnp.int32)]
    ).reshape(NW * ENB, EB)
    dst2 = jnp.concatenate(
        [edge_index[1].astype(jnp.int32), jnp.full((epad,), N, jnp.int32)]
    ).reshape(NW * ENB, EB)

    nl = edge_label_index.shape[1]
    lpad = LPAD - nl
    sl2 = jnp.concatenate(
        [edge_label_index[0].astype(jnp.int32), jnp.zeros((lpad,), jnp.int32)]
    ).reshape(NW * LNB, LB)
    dl2 = jnp.concatenate(
        [edge_label_index[1].astype(jnp.int32), jnp.zeros((lpad,), jnp.int32)]
    ).reshape(NW * LNB, LB)

    deg = _degrees(dst2)
    hlo, hhi, dinv = _t1(deg, embedding, W1)
    p = _aggregate_ab(hlo, hhi, src2, dst2)
    hs2 = _t2(p, hlo, hhi, dinv, b1[None, :], W2)
    q = _aggregate(hs2, src2, dst2)
    z = _t3(q, hs2, dinv, b2[None, :])
    scores = _decode(z, sl2, dl2)
    return scores[:nl]
